# baseline (device time: 33631 ns/iter reference)
import jax
import jax.numpy as jnp
from jax import lax
from jax.experimental import pallas as pl
from jax.experimental.pallas import tpu as pltpu


def kernel(O, Wo):
    B, S, H, D = O.shape
    K = H * D
    N = Wo.shape[1]
    S_half = S // 2

    O2 = O.reshape(B, S, K)

    def body(o_ref, w_ref, out_ref, send_buf, recv_buf, send_sem, recv_sem):
        my_x = lax.axis_index("x")
        my_y = lax.axis_index("y")
        my_z = lax.axis_index("z")
        peer = (1 - my_x, my_y, my_z)

        barrier = pltpu.get_barrier_semaphore()
        pl.semaphore_signal(
            barrier, inc=1, device_id=peer, device_id_type=pl.DeviceIdType.MESH
        )
        pl.semaphore_wait(barrier, 1)

        w = w_ref[...].astype(jnp.bfloat16)
        peer_start = (1 - my_x) * S_half
        my_start = my_x * S_half

        for b in range(B):
            o_b = o_ref[b, pl.ds(peer_start, S_half), :].astype(jnp.bfloat16)
            send_buf[b] = jnp.dot(
                o_b, w, preferred_element_type=jnp.float32
            ).astype(jnp.bfloat16)

        rdma = pltpu.make_async_remote_copy(
            src_ref=send_buf,
            dst_ref=recv_buf,
            send_sem=send_sem,
            recv_sem=recv_sem,
            device_id=peer,
            device_id_type=pl.DeviceIdType.MESH,
        )
        rdma.start()

        for b in range(B):
            o_b = o_ref[b, pl.ds(my_start, S_half), :].astype(jnp.bfloat16)
            out_ref[b] = jnp.dot(o_b, w, preferred_element_type=jnp.float32)

        rdma.wait()

        for b in range(B):
            out_ref[b] = out_ref[b] + recv_buf[b].astype(jnp.float32)

    return pl.pallas_call(
        body,
        out_shape=jax.ShapeDtypeStruct((B, S_half, N), jnp.float32),
        in_specs=[
            pl.BlockSpec(memory_space=pltpu.VMEM),
            pl.BlockSpec(memory_space=pltpu.VMEM),
        ],
        out_specs=pl.BlockSpec(memory_space=pltpu.VMEM),
        scratch_shapes=[
            pltpu.VMEM((B, S_half, N), jnp.bfloat16),
            pltpu.VMEM((B, S_half, N), jnp.bfloat16),
            pltpu.SemaphoreType.DMA,
            pltpu.SemaphoreType.DMA,
        ],
        compiler_params=pltpu.CompilerParams(collective_id=0),
    )(O2, Wo)


# device time: 32460 ns/iter; 1.0361x vs baseline; 1.0361x over previous
import jax
import jax.numpy as jnp
from jax import lax
from jax.experimental import pallas as pl
from jax.experimental.pallas import tpu as pltpu


def kernel(O, Wo):
    B, S, H, D = O.shape
    K = H * D
    N = Wo.shape[1]
    S_half = S // 2

    O2 = O.reshape(B, S, K)

    def body(o_ref, w_ref, out_ref, send_buf, recv_buf, send_sem, recv_sem):
        my_x = lax.axis_index("x")
        my_y = lax.axis_index("y")
        my_z = lax.axis_index("z")
        peer = (1 - my_x, my_y, my_z)

        barrier = pltpu.get_barrier_semaphore()
        pl.semaphore_signal(
            barrier, inc=1, device_id=peer, device_id_type=pl.DeviceIdType.MESH
        )
        pl.semaphore_wait(barrier, 1)

        w = w_ref[...].astype(jnp.bfloat16)
        peer_start = (1 - my_x) * S_half
        my_start = my_x * S_half

        rdmas = []
        for b in range(B):
            o_b = o_ref[b, pl.ds(peer_start, S_half), :].astype(jnp.bfloat16)
            send_buf[b] = jnp.dot(
                o_b, w, preferred_element_type=jnp.float32
            ).astype(jnp.bfloat16)
            rdma = pltpu.make_async_remote_copy(
                src_ref=send_buf.at[b],
                dst_ref=recv_buf.at[b],
                send_sem=send_sem.at[b],
                recv_sem=recv_sem.at[b],
                device_id=peer,
                device_id_type=pl.DeviceIdType.MESH,
            )
            rdma.start()
            rdmas.append(rdma)

        for b in range(B):
            o_b = o_ref[b, pl.ds(my_start, S_half), :].astype(jnp.bfloat16)
            out_ref[b] = jnp.dot(o_b, w, preferred_element_type=jnp.float32)

        for b in range(B):
            rdmas[b].wait()
            out_ref[b] = out_ref[b] + recv_buf[b].astype(jnp.float32)

    return pl.pallas_call(
        body,
        out_shape=jax.ShapeDtypeStruct((B, S_half, N), jnp.float32),
        in_specs=[
            pl.BlockSpec(memory_space=pltpu.VMEM),
            pl.BlockSpec(memory_space=pltpu.VMEM),
        ],
        out_specs=pl.BlockSpec(memory_space=pltpu.VMEM),
        scratch_shapes=[
            pltpu.VMEM((B, S_half, N), jnp.bfloat16),
            pltpu.VMEM((B, S_half, N), jnp.bfloat16),
            pltpu.SemaphoreType.DMA((B,)),
            pltpu.SemaphoreType.DMA((B,)),
        ],
        compiler_params=pltpu.CompilerParams(collective_id=0),
    )(O2, Wo)


# device time: 32401 ns/iter; 1.0380x vs baseline; 1.0018x over previous
import jax
import jax.numpy as jnp
from jax import lax
from jax.experimental import pallas as pl
from jax.experimental.pallas import tpu as pltpu


def kernel(O, Wo):
    B, S, H, D = O.shape
    K = H * D
    N = Wo.shape[1]
    S_half = S // 2

    O2 = jnp.swapaxes(O.reshape(B, S, K), 1, 2)

    def body(o_ref, w_ref, out_ref, send_buf, recv_buf, send_sem, recv_sem):
        my_x = lax.axis_index("x")
        my_y = lax.axis_index("y")
        my_z = lax.axis_index("z")
        peer = (1 - my_x, my_y, my_z)

        barrier = pltpu.get_barrier_semaphore()
        pl.semaphore_signal(
            barrier, inc=1, device_id=peer, device_id_type=pl.DeviceIdType.MESH
        )
        pl.semaphore_wait(barrier, 1)

        w = w_ref[...].astype(jnp.bfloat16)
        peer_start = (1 - my_x) * S_half
        my_start = my_x * S_half

        def matmul_shalf(b, start):
            o_b = o_ref[b, :, pl.ds(start, S_half)].astype(jnp.bfloat16)
            return lax.dot_general(
                o_b, w,
                dimension_numbers=(((0,), (0,)), ((), ())),
                preferred_element_type=jnp.float32,
            )

        rdmas = []
        for b in range(B):
            send_buf[b] = matmul_shalf(b, peer_start).astype(jnp.bfloat16)
            rdma = pltpu.make_async_remote_copy(
                src_ref=send_buf.at[b],
                dst_ref=recv_buf.at[b],
                send_sem=send_sem.at[b],
                recv_sem=recv_sem.at[b],
                device_id=peer,
                device_id_type=pl.DeviceIdType.MESH,
            )
            rdma.start()
            rdmas.append(rdma)

        for b in range(B):
            out_ref[b] = matmul_shalf(b, my_start)

        for b in range(B):
            rdmas[b].wait()
            out_ref[b] = out_ref[b] + recv_buf[b].astype(jnp.float32)

    return pl.pallas_call(
        body,
        out_shape=jax.ShapeDtypeStruct((B, S_half, N), jnp.float32),
        in_specs=[
            pl.BlockSpec(memory_space=pltpu.VMEM),
            pl.BlockSpec(memory_space=pltpu.VMEM),
        ],
        out_specs=pl.BlockSpec(memory_space=pltpu.VMEM),
        scratch_shapes=[
            pltpu.VMEM((B, S_half, N), jnp.bfloat16),
            pltpu.VMEM((B, S_half, N), jnp.bfloat16),
            pltpu.SemaphoreType.DMA((B,)),
            pltpu.SemaphoreType.DMA((B,)),
        ],
        compiler_params=pltpu.CompilerParams(collective_id=0),
    )(O2, Wo)
